# baseline (speedup 1.0000x reference)
import functools

import jax
import jax.numpy as jnp
from jax.experimental import pallas as pl
from jax.experimental.pallas import tpu as pltpu

_F = 128
_F8 = jnp.float8_e4m3fn


def _gcn_body(x_ref, adj_ref, W0_ref, Wn_ref, b_ref, fcW_ref, fcb_ref,
              out_ref, adj8_ref, h_ref, z_ref, *, nj0, r0, nsub, rbig, nl):
    t = pl.program_id(0)
    is_l0 = t < nj0
    layer = jnp.where(is_l0, 0, (t - nj0) // nsub + 1)
    sub = jnp.where(is_l0, 0, (t - nj0) % nsub)
    cur = jax.lax.rem(layer, 2)
    nxt = jax.lax.rem(layer + 1, 2)

    @pl.when(t == 0)
    def _():
        xb = x_ref[...]
        z_ref[0, 0] = jax.lax.dot(
            xb, W0_ref[0, 0], preferred_element_type=jnp.float32).astype(_F8)
        z_ref[0, 1] = jax.lax.dot(
            xb, W0_ref[0, 1], preferred_element_type=jnp.float32).astype(_F8)

    def _znext(rows0, rr, hb):
        z_ref[nxt, 0, pl.ds(rows0, rr), :] = jax.lax.dot(
            hb, Wn_ref[0, 0], preferred_element_type=jnp.float32).astype(_F8)
        z_ref[nxt, 1, pl.ds(rows0, rr), :] = jax.lax.dot(
            hb, Wn_ref[0, 1], preferred_element_type=jnp.float32).astype(_F8)

    @pl.when(is_l0)
    def _():
        row0 = pl.multiple_of(t * r0, r0)
        adj8_ref[:, pl.ds(row0, r0), :] = (adj_ref[...] * 4096.0).astype(_F8)
        acc = jax.lax.dot(adj8_ref[0, pl.ds(row0, r0), :], z_ref[0, 0],
                          preferred_element_type=jnp.float32)
        acc = acc + jax.lax.dot(adj8_ref[1, pl.ds(row0, r0), :], z_ref[0, 1],
                                preferred_element_type=jnp.float32)
        hn = jnp.tanh(acc * (1.0 / 4096.0) + b_ref[0, 0][None, :])
        h_ref[pl.ds(row0, r0), :] = hn
        _znext(row0, r0, hn.astype(jnp.bfloat16))

    @pl.when(jnp.logical_not(is_l0))
    def _():
        row0 = pl.multiple_of(sub * rbig, rbig)
        acc = jax.lax.dot(adj8_ref[0, pl.ds(row0, rbig), :], z_ref[cur, 0],
                          preferred_element_type=jnp.float32)
        acc = acc + jax.lax.dot(adj8_ref[1, pl.ds(row0, rbig), :],
                                z_ref[cur, 1],
                                preferred_element_type=jnp.float32)
        acc = acc * (1.0 / 4096.0)
        resid = jnp.logical_or(jnp.logical_or(layer == 1, layer == 3),
                               layer >= 5)
        hcur = jnp.where(resid, h_ref[pl.ds(row0, rbig), :], 0.0)
        hn = jnp.tanh(acc + b_ref[0, 0][None, :]) + hcur
        h_ref[pl.ds(row0, rbig), :] = hn

        @pl.when(layer < nl - 1)
        def _():
            _znext(row0, rbig, hn.astype(jnp.bfloat16))

    @pl.when(t == nj0 + (nl - 1) * nsub - 1)
    def _():
        hf = h_ref[...].astype(jnp.bfloat16)
        t1 = jnp.tanh(jax.lax.dot(hf, fcW_ref[0],
                                  preferred_element_type=jnp.float32)
                      + fcb_ref[0, 0][None, :])
        t2 = jnp.tanh(jax.lax.dot(t1.astype(jnp.bfloat16), fcW_ref[1],
                                  preferred_element_type=jnp.float32)
                      + fcb_ref[1, 0][None, :]) + t1
        t3 = jnp.tanh(jax.lax.dot(t2.astype(jnp.bfloat16), fcW_ref[2],
                                  preferred_element_type=jnp.float32)
                      + fcb_ref[2, 0][None, :])
        out_ref[...] = (t3 + 1.0) * 0.5


def kernel(x, adj_list, params):
    gcW, gcb, fcW, fcb = params
    n, f_in = x.shape
    f = _F
    nl = len(gcW)

    Ws = jnp.stack([
        jnp.pad(w, ((0, 0), (0, f - w.shape[1]), (0, f - w.shape[2])))
        for w in gcW
    ]).astype(jnp.bfloat16)
    bs = jnp.stack([jnp.pad(b, (0, f - b.shape[0]))
                    for b in gcb])[:, None, :]
    fWs = jnp.stack([
        jnp.pad(w, ((0, f - w.shape[0]), (0, f - w.shape[1]))) for w in fcW
    ]).astype(jnp.bfloat16)
    fbs = jnp.stack([jnp.pad(b, (0, f - b.shape[0]))
                     for b in fcb])[:, None, :]
    xp = jnp.pad(x, ((0, 0), (0, f - f_in))).astype(jnp.bfloat16)

    r0 = 128 if n % 128 == 0 else n
    nj0 = n // r0
    rbig = 2048 if n % 2048 == 0 else n
    nsub = n // rbig
    nsteps = nj0 + (nl - 1) * nsub

    def _layer_of(t):
        return jnp.where(t < nj0, 0, (t - nj0) // nsub + 1)

    out = pl.pallas_call(
        functools.partial(_gcn_body, nj0=nj0, r0=r0, nsub=nsub, rbig=rbig,
                          nl=nl),
        grid=(nsteps,),
        in_specs=[
            pl.BlockSpec((n, f), lambda t: (0, 0)),
            pl.BlockSpec((2, r0, n),
                         lambda t: (0, jnp.where(t < nj0, t, nj0 - 1), 0)),
            pl.BlockSpec((1, 2, f, f), lambda t: (0, 0, 0, 0)),
            pl.BlockSpec((1, 2, f, f),
                         lambda t: (jnp.minimum(_layer_of(t) + 1, nl - 1),
                                    0, 0, 0)),
            pl.BlockSpec((1, 1, f), lambda t: (_layer_of(t), 0, 0)),
            pl.BlockSpec((3, f, f), lambda t: (0, 0, 0)),
            pl.BlockSpec((3, 1, f), lambda t: (0, 0, 0)),
        ],
        out_specs=pl.BlockSpec((n, f), lambda t: (0, 0)),
        out_shape=jax.ShapeDtypeStruct((n, f), jnp.float32),
        scratch_shapes=[
            pltpu.VMEM((2, n, n), _F8),
            pltpu.VMEM((n, f), jnp.float32),
            pltpu.VMEM((2, 2, n, f), _F8),
        ],
        compiler_params=pltpu.CompilerParams(
            dimension_semantics=("arbitrary",),
            vmem_limit_bytes=62 * 1024 * 1024,
        ),
    )(xp, adj_list, Ws, Ws, bs, fWs, fbs)
    return out[:, :1]


# speedup vs baseline: 1.0502x; 1.0502x over previous
import functools

import jax
import jax.numpy as jnp
from jax.experimental import pallas as pl
from jax.experimental.pallas import tpu as pltpu

_F = 128
_F8 = jnp.float8_e4m3fn


def _gcn_body(x_ref, adj_ref, W_ref, b_ref, fcW_ref, fcb_ref, out_ref,
              adj8_ref, h_ref, z_ref, *, nj0, r0, nsub, rbig, nl):
    t = pl.program_id(0)
    is_l0 = t < nj0
    layer = jnp.where(is_l0, 0, (t - nj0) // nsub + 1)
    sub = jnp.where(is_l0, 0, (t - nj0) % nsub)
    layer_start = jnp.logical_or(t == 0,
                                 jnp.logical_and(jnp.logical_not(is_l0),
                                                 sub == 0))

    @pl.when(layer_start)
    def _():
        hb = jnp.where(t == 0, x_ref[...].astype(jnp.float32),
                       h_ref[...]).astype(jnp.bfloat16)
        z_ref[0] = jax.lax.dot(
            hb, W_ref[0, 0], preferred_element_type=jnp.float32).astype(_F8)
        z_ref[1] = jax.lax.dot(
            hb, W_ref[0, 1], preferred_element_type=jnp.float32).astype(_F8)

    @pl.when(is_l0)
    def _():
        row0 = pl.multiple_of(t * r0, r0)
        adj8_ref[:, pl.ds(row0, r0), :] = (adj_ref[...] * 4096.0).astype(_F8)
        acc = jax.lax.dot(adj8_ref[0, pl.ds(row0, r0), :], z_ref[0],
                          preferred_element_type=jnp.float32)
        acc = acc + jax.lax.dot(adj8_ref[1, pl.ds(row0, r0), :], z_ref[1],
                                preferred_element_type=jnp.float32)
        h_ref[pl.ds(row0, r0), :] = jnp.tanh(
            acc * (1.0 / 4096.0) + b_ref[0, 0][None, :])

    @pl.when(jnp.logical_not(is_l0))
    def _():
        row0 = pl.multiple_of(sub * rbig, rbig)
        acc = jax.lax.dot(adj8_ref[0, pl.ds(row0, rbig), :], z_ref[0],
                          preferred_element_type=jnp.float32)
        acc = acc + jax.lax.dot(adj8_ref[1, pl.ds(row0, rbig), :], z_ref[1],
                                preferred_element_type=jnp.float32)
        acc = acc * (1.0 / 4096.0)
        resid = jnp.logical_or(jnp.logical_or(layer == 1, layer == 3),
                               layer >= 5)
        hcur = jnp.where(resid, h_ref[pl.ds(row0, rbig), :], 0.0)
        h_ref[pl.ds(row0, rbig), :] = (
            jnp.tanh(acc + b_ref[0, 0][None, :]) + hcur)

    @pl.when(t == nj0 + (nl - 1) * nsub - 1)
    def _():
        hf = h_ref[...].astype(jnp.bfloat16)
        t1 = jnp.tanh(jax.lax.dot(hf, fcW_ref[0],
                                  preferred_element_type=jnp.float32)
                      + fcb_ref[0, 0][None, :])
        t2 = jnp.tanh(jax.lax.dot(t1.astype(jnp.bfloat16), fcW_ref[1],
                                  preferred_element_type=jnp.float32)
                      + fcb_ref[1, 0][None, :]) + t1
        t3 = jnp.tanh(jax.lax.dot(t2.astype(jnp.bfloat16), fcW_ref[2],
                                  preferred_element_type=jnp.float32)
                      + fcb_ref[2, 0][None, :])
        out_ref[...] = (t3 + 1.0) * 0.5


def kernel(x, adj_list, params):
    gcW, gcb, fcW, fcb = params
    n, f_in = x.shape
    f = _F
    nl = len(gcW)

    Ws = jnp.stack([
        jnp.pad(w, ((0, 0), (0, f - w.shape[1]), (0, f - w.shape[2])))
        for w in gcW
    ]).astype(jnp.bfloat16)
    bs = jnp.stack([jnp.pad(b, (0, f - b.shape[0]))
                    for b in gcb])[:, None, :]
    fWs = jnp.stack([
        jnp.pad(w, ((0, f - w.shape[0]), (0, f - w.shape[1]))) for w in fcW
    ]).astype(jnp.bfloat16)
    fbs = jnp.stack([jnp.pad(b, (0, f - b.shape[0]))
                     for b in fcb])[:, None, :]
    xp = jnp.pad(x, ((0, 0), (0, f - f_in))).astype(jnp.bfloat16)

    r0 = 128 if n % 128 == 0 else n
    nj0 = n // r0
    rbig = 1024 if n % 1024 == 0 else n
    nsub = n // rbig
    nsteps = nj0 + (nl - 1) * nsub

    def _layer_of(t):
        return jnp.where(t < nj0, 0, (t - nj0) // nsub + 1)

    out = pl.pallas_call(
        functools.partial(_gcn_body, nj0=nj0, r0=r0, nsub=nsub, rbig=rbig,
                          nl=nl),
        grid=(nsteps,),
        in_specs=[
            pl.BlockSpec((n, f), lambda t: (0, 0)),
            pl.BlockSpec((2, r0, n),
                         lambda t: (0, jnp.where(t < nj0, t, nj0 - 1), 0)),
            pl.BlockSpec((1, 2, f, f), lambda t: (_layer_of(t), 0, 0, 0)),
            pl.BlockSpec((1, 1, f), lambda t: (_layer_of(t), 0, 0)),
            pl.BlockSpec((3, f, f), lambda t: (0, 0, 0)),
            pl.BlockSpec((3, 1, f), lambda t: (0, 0, 0)),
        ],
        out_specs=pl.BlockSpec((n, f), lambda t: (0, 0)),
        out_shape=jax.ShapeDtypeStruct((n, f), jnp.float32),
        scratch_shapes=[
            pltpu.VMEM((2, n, n), _F8),
            pltpu.VMEM((n, f), jnp.float32),
            pltpu.VMEM((2, n, f), _F8),
        ],
        compiler_params=pltpu.CompilerParams(
            dimension_semantics=("arbitrary",),
            vmem_limit_bytes=62 * 1024 * 1024,
        ),
    )(xp, adj_list, Ws, bs, fWs, fbs)
    return out[:, :1]
